# initial kernel scaffold (unmeasured)
import jax
import jax.numpy as jnp
from jax import lax
from jax.experimental import pallas as pl
from jax.experimental.pallas import tpu as pltpu

N_Z = 4
SCALE = 64 ** -0.5

_CompilerParams = getattr(pltpu, "CompilerParams", None) or getattr(
    pltpu, "TPUCompilerParams"
)


def kernel(Q, K, V):
    b, s_per, h, d = Q.shape

    def body(q_ref, k_ref, v_ref, out_ref, k_all, v_all,
             k_send, k_recv, v_send, v_recv):
        my_x = lax.axis_index("x")
        my_y = lax.axis_index("y")
        my_z = lax.axis_index("z")
        fwd = (my_z + 1) % N_Z
        bwd = (my_z - 1) % N_Z

        barrier = pltpu.get_barrier_semaphore()
        for nbr in (fwd, bwd):
            pl.semaphore_signal(
                barrier, inc=1,
                device_id=(my_x, my_y, nbr),
                device_id_type=pl.DeviceIdType.MESH,
            )
        pl.semaphore_wait(barrier, 2)

        k_all[my_z] = k_ref[...].astype(jnp.bfloat16)
        v_all[my_z] = v_ref[...].astype(jnp.bfloat16)

        for hop in range(N_Z - 1):
            ks = (my_z - hop) % N_Z
            vs = (my_z + hop) % N_Z
            k_rdma = pltpu.make_async_remote_copy(
                src_ref=k_all.at[ks],
                dst_ref=k_all.at[ks],
                send_sem=k_send.at[hop],
                recv_sem=k_recv.at[hop],
                device_id=(my_x, my_y, fwd),
                device_id_type=pl.DeviceIdType.MESH,
            )
            v_rdma = pltpu.make_async_remote_copy(
                src_ref=v_all.at[vs],
                dst_ref=v_all.at[vs],
                send_sem=v_send.at[hop],
                recv_sem=v_recv.at[hop],
                device_id=(my_x, my_y, bwd),
                device_id_type=pl.DeviceIdType.MESH,
            )
            k_rdma.start()
            v_rdma.start()
            k_rdma.wait()
            v_rdma.wait()

        qs = (q_ref[...] * SCALE).astype(jnp.bfloat16)
        for bb in range(b):
            for hh in range(h):
                q = qs[bb, :, hh, :]
                k = k_all[:, bb, :, hh, :].reshape(N_Z * s_per, d)
                v = v_all[:, bb, :, hh, :].reshape(N_Z * s_per, d)
                s_mat = lax.dot_general(
                    q, k, (((1,), (1,)), ((), ())),
                    preferred_element_type=jnp.float32,
                )
                m = jnp.max(s_mat, axis=1, keepdims=True)
                p = jnp.exp(s_mat - m)
                denom = jnp.sum(p, axis=1, keepdims=True)
                o = lax.dot_general(
                    p.astype(jnp.bfloat16), v, (((1,), (1,)), ((), ())),
                    preferred_element_type=jnp.float32,
                )
                out_ref[bb, :, hh, :] = o / denom

    return pl.pallas_call(
        body,
        out_shape=jax.ShapeDtypeStruct((b, s_per, h, d), jnp.float32),
        in_specs=[pl.BlockSpec(memory_space=pltpu.VMEM)] * 3,
        out_specs=pl.BlockSpec(memory_space=pltpu.VMEM),
        scratch_shapes=[
            pltpu.VMEM((N_Z, b, s_per, h, d), jnp.bfloat16),
            pltpu.VMEM((N_Z, b, s_per, h, d), jnp.bfloat16),
            pltpu.SemaphoreType.DMA((N_Z - 1,)),
            pltpu.SemaphoreType.DMA((N_Z - 1,)),
            pltpu.SemaphoreType.DMA((N_Z - 1,)),
            pltpu.SemaphoreType.DMA((N_Z - 1,)),
        ],
        compiler_params=_CompilerParams(collective_id=0),
    )(Q, K, V)


# baseline (device time: 203716 ns/iter reference)
import jax
import jax.numpy as jnp
from jax import lax
from jax.experimental import pallas as pl
from jax.experimental.pallas import tpu as pltpu

N_Z = 4
SCALE = 64 ** -0.5

_CompilerParams = getattr(pltpu, "CompilerParams", None) or getattr(
    pltpu, "TPUCompilerParams"
)


def kernel(Q, K, V):
    b, s_per, h, d = Q.shape

    Qh = jnp.transpose(Q * SCALE, (0, 2, 1, 3)).astype(jnp.bfloat16)
    Kh = jnp.transpose(K, (0, 2, 1, 3)).astype(jnp.bfloat16)
    Vh = jnp.transpose(V, (0, 2, 1, 3)).astype(jnp.bfloat16)

    def body(q_ref, k_ref, v_ref, out_ref, kv_all, send_sems, recv_sems):
        my_x = lax.axis_index("x")
        my_y = lax.axis_index("y")
        my_z = lax.axis_index("z")

        barrier = pltpu.get_barrier_semaphore()
        for off in range(1, N_Z):
            pl.semaphore_signal(
                barrier, inc=1,
                device_id=(my_x, my_y, (my_z + off) % N_Z),
                device_id_type=pl.DeviceIdType.MESH,
            )
        pl.semaphore_wait(barrier, N_Z - 1)

        kv_all[my_z, 0] = k_ref[...]
        kv_all[my_z, 1] = v_ref[...]

        rdmas = []
        for off in range(1, N_Z):
            rdma = pltpu.make_async_remote_copy(
                src_ref=kv_all.at[my_z],
                dst_ref=kv_all.at[my_z],
                send_sem=send_sems.at[off - 1],
                recv_sem=recv_sems.at[off - 1],
                device_id=(my_x, my_y, (my_z + off) % N_Z),
                device_id_type=pl.DeviceIdType.MESH,
            )
            rdma.start()
            rdmas.append(rdma)
        for rdma in rdmas:
            rdma.wait()

        def attn_block(i, carry):
            bb = i // h
            hh = i % h
            q = q_ref[bb, hh]
            k = jnp.concatenate(
                [kv_all[zz, 0, bb, hh] for zz in range(N_Z)], axis=0
            )
            v = jnp.concatenate(
                [kv_all[zz, 1, bb, hh] for zz in range(N_Z)], axis=0
            )
            s_mat = lax.dot_general(
                q, k, (((1,), (1,)), ((), ())),
                preferred_element_type=jnp.float32,
            )
            m = jnp.max(s_mat, axis=1, keepdims=True)
            p = jnp.exp(s_mat - m)
            denom = jnp.sum(p, axis=1, keepdims=True)
            o = lax.dot_general(
                p.astype(jnp.bfloat16), v, (((1,), (0,)), ((), ())),
                preferred_element_type=jnp.float32,
            )
            out_ref[bb, hh] = o / denom
            return carry

        lax.fori_loop(0, b * h, attn_block, 0)

    out_hmajor = pl.pallas_call(
        body,
        out_shape=jax.ShapeDtypeStruct((b, h, s_per, d), jnp.float32),
        in_specs=[pl.BlockSpec(memory_space=pltpu.VMEM)] * 3,
        out_specs=pl.BlockSpec(memory_space=pltpu.VMEM),
        scratch_shapes=[
            pltpu.VMEM((N_Z, 2, b, h, s_per, d), jnp.bfloat16),
            pltpu.SemaphoreType.DMA((N_Z - 1,)),
            pltpu.SemaphoreType.DMA((N_Z - 1,)),
        ],
        compiler_params=_CompilerParams(
            collective_id=0, vmem_limit_bytes=100 * 1024 * 1024
        ),
    )(Qh, Kh, Vh)

    return jnp.transpose(out_hmajor, (0, 2, 1, 3))


# device time: 118768 ns/iter; 1.7152x vs baseline; 1.7152x over previous
import jax
import jax.numpy as jnp
from jax import lax
from jax.experimental import pallas as pl
from jax.experimental.pallas import tpu as pltpu

N_Z = 4
SCALE = 64 ** -0.5

_CompilerParams = getattr(pltpu, "CompilerParams", None) or getattr(
    pltpu, "TPUCompilerParams"
)


def _pack(x):
    b, s, h, d = x.shape
    x = jnp.transpose(x, (0, 2, 1, 3))
    x = x.reshape(b, h // 2, 2, s, d)
    x = jnp.transpose(x, (0, 1, 3, 2, 4))
    return x.reshape(b, h // 2, s, 2 * d)


def kernel(Q, K, V):
    b, s_per, h, d = Q.shape
    hp = h // 2
    d2 = 2 * d

    Qp = _pack(Q * SCALE).astype(jnp.bfloat16)
    Kp = _pack(K).astype(jnp.bfloat16)
    Vp = _pack(V).astype(jnp.bfloat16)

    def body(q_ref, k_ref, v_ref, out_ref, kv_all, send_sems, recv_sems):
        my_x = lax.axis_index("x")
        my_y = lax.axis_index("y")
        my_z = lax.axis_index("z")

        barrier = pltpu.get_barrier_semaphore()
        for off in range(1, N_Z):
            pl.semaphore_signal(
                barrier, inc=1,
                device_id=(my_x, my_y, (my_z + off) % N_Z),
                device_id_type=pl.DeviceIdType.MESH,
            )
        pl.semaphore_wait(barrier, N_Z - 1)

        kv_all[my_z, 0] = k_ref[...]
        kv_all[my_z, 1] = v_ref[...]

        rdmas = []
        for off in range(1, N_Z):
            rdma = pltpu.make_async_remote_copy(
                src_ref=kv_all.at[my_z],
                dst_ref=kv_all.at[my_z],
                send_sem=send_sems.at[off - 1],
                recv_sem=recv_sems.at[off - 1],
                device_id=(my_x, my_y, (my_z + off) % N_Z),
                device_id_type=pl.DeviceIdType.MESH,
            )
            rdma.start()
            rdmas.append(rdma)
        for rdma in rdmas:
            rdma.wait()

        def attn_block(i, carry):
            bb = i // hp
            pp = i % hp
            q2 = q_ref[bb, pp]
            k2 = jnp.concatenate(
                [kv_all[zz, 0, bb, pp] for zz in range(N_Z)], axis=0
            )
            v2 = jnp.concatenate(
                [kv_all[zz, 1, bb, pp] for zz in range(N_Z)], axis=0
            )
            halves = []
            for half in range(2):
                sl = slice(half * d, (half + 1) * d)
                s_mat = lax.dot_general(
                    q2[:, sl], k2[:, sl], (((1,), (1,)), ((), ())),
                    preferred_element_type=jnp.float32,
                )
                m = jnp.max(s_mat, axis=1, keepdims=True)
                p = jnp.exp(s_mat - m)
                denom = jnp.sum(p, axis=1, keepdims=True)
                o = lax.dot_general(
                    p.astype(jnp.bfloat16), v2[:, sl],
                    (((1,), (0,)), ((), ())),
                    preferred_element_type=jnp.float32,
                )
                halves.append(o / denom)
            out_ref[bb, pp] = jnp.concatenate(halves, axis=1)
            return carry

        lax.fori_loop(0, b * hp, attn_block, 0)

    out_p = pl.pallas_call(
        body,
        out_shape=jax.ShapeDtypeStruct((b, hp, s_per, d2), jnp.float32),
        in_specs=[pl.BlockSpec(memory_space=pltpu.VMEM)] * 3,
        out_specs=pl.BlockSpec(memory_space=pltpu.VMEM),
        scratch_shapes=[
            pltpu.VMEM((N_Z, 2, b, hp, s_per, d2), jnp.bfloat16),
            pltpu.SemaphoreType.DMA((N_Z - 1,)),
            pltpu.SemaphoreType.DMA((N_Z - 1,)),
        ],
        compiler_params=_CompilerParams(
            collective_id=0, vmem_limit_bytes=100 * 1024 * 1024
        ),
    )(Qp, Kp, Vp)

    out_p = out_p.reshape(b, hp, s_per, 2, d)
    out_p = jnp.transpose(out_p, (0, 2, 1, 3, 4))
    return out_p.reshape(b, s_per, h, d)
